# Initial kernel scaffold; baseline (speedup 1.0000x reference)
import jax
import jax.numpy as jnp
from jax.experimental import pallas as pl


def kernel(img_render_points, img_ref):
    raise NotImplementedError("write your pallas kernel here")


# speedup vs baseline: 1.2986x; 1.2986x over previous
import functools

import jax
import jax.numpy as jnp
from jax import lax
from jax.experimental import pallas as pl
from jax.experimental.pallas import tpu as pltpu
from jax.experimental.pallas import tpu_sc as plsc

_NUM_CORES = 2
_NUM_SUBCORES = 16
_LANES = 16
_NW = _NUM_CORES * _NUM_SUBCORES


@functools.lru_cache(maxsize=None)
def _sc_min_d2(h, w, n):
    rows_w = h // _NW
    groups = w // _LANES
    pts_w = n // _NW
    assert rows_w * _NW == h and groups * _LANES == w and pts_w * _NW == n
    assert pts_w == _LANES

    mesh = plsc.VectorSubcoreMesh(core_axis_name="c", subcore_axis_name="s")

    @functools.partial(
        pl.kernel,
        out_type=(
            jax.ShapeDtypeStruct((_NW, h // _NW, w), jnp.float32),
            jax.ShapeDtypeStruct((_NW, n // _NW), jnp.float32),
        ),
        mesh=mesh,
        scratch_types=[
            pltpu.VMEM((n,), jnp.float32),
            pltpu.VMEM((n,), jnp.float32),
            pltpu.VMEM((rows_w, w), jnp.float32),
            pltpu.VMEM((pts_w,), jnp.float32),
        ],
    )
    def sc_k(py_hbm, px_hbm, img_out, pt_out, py_v, px_v, rows_v, ptd_v):
        cid = lax.axis_index("c")
        sid = lax.axis_index("s")
        wid = sid * _NUM_CORES + cid

        pltpu.sync_copy(py_hbm, py_v)
        pltpu.sync_copy(px_hbm, px_v)

        p0 = wid * pts_w
        pyv = py_v[pl.ds(p0, _LANES)]
        pxv = px_v[pl.ds(p0, _LANES)]

        def lattice_d2(v, hi):
            idx = (v + 0.5).astype(jnp.int32)
            idx = jnp.minimum(jnp.maximum(idx, 0), hi)
            d = v - idx.astype(jnp.float32)
            return d * d

        ptd_v[...] = lattice_d2(pyv, h - 1) + lattice_d2(pxv, w - 1)
        pltpu.sync_copy(ptd_v, pt_out.at[wid])

        row0 = wid * rows_w
        iota = lax.iota(jnp.int32, _LANES).astype(jnp.float32)
        xvecs = [iota + float(g * _LANES) for g in range(groups)]
        init = tuple(jnp.full((_LANES,), 1e30, jnp.float32) for _ in range(groups))

        chunks = n // _LANES
        for r in range(rows_w):
            yf = (row0 + r).astype(jnp.float32)

            def body(jc, accs, yf=yf):
                j0 = jc * _LANES
                pyv = py_v[pl.ds(j0, _LANES)]
                pxv = px_v[pl.ds(j0, _LANES)]
                out = list(accs)
                for k in range(_LANES):
                    pyj = pyv[k]
                    pxj = pxv[k]
                    dy = yf - pyj
                    dy2 = dy * dy
                    for g in range(groups):
                        dx = xvecs[g] - pxj
                        out[g] = jnp.minimum(out[g], dx * dx + dy2)
                return tuple(out)

            accs = lax.fori_loop(0, chunks, body, init)
            for g in range(groups):
                rows_v[r, pl.ds(g * _LANES, _LANES)] = accs[g]

        pltpu.sync_copy(rows_v, img_out.at[wid])

    return sc_k


def _tc_finish(img_ref, pt_ref, out_ref):
    total = (jnp.sum(jnp.sqrt(img_ref[...]))
             + jnp.sum(jnp.sqrt(pt_ref[...])))
    out_ref[...] = jnp.broadcast_to(total, (1, 1))


def kernel(img_render_points, img_ref):
    pts = img_render_points.reshape(-1, 2).astype(jnp.float32)
    n = pts.shape[0]
    h, w = img_ref.shape
    py = pts[:, 0]
    px = pts[:, 1]

    img_d2, pt_d2 = _sc_min_d2(h, w, n)(py, px)

    loss = pl.pallas_call(
        _tc_finish,
        out_shape=jax.ShapeDtypeStruct((1, 1), jnp.float32),
    )(img_d2, pt_d2.reshape(n // 128, 128))
    return loss[0, 0]


# speedup vs baseline: 1.7856x; 1.3750x over previous
import functools

import jax
import jax.numpy as jnp
from jax import lax
from jax.experimental import pallas as pl
from jax.experimental.pallas import tpu as pltpu
from jax.experimental.pallas import tpu_sc as plsc

_NUM_CORES = 2
_NUM_SUBCORES = 16
_LANES = 16
_NW = _NUM_CORES * _NUM_SUBCORES


@functools.lru_cache(maxsize=None)
def _sc_min_d2(h, w, n):
    rows_w = h // _NW
    groups = w // _LANES
    pts_w = n // _NW
    assert rows_w * _NW == h and groups * _LANES == w and pts_w * _NW == n
    assert pts_w == _LANES

    mesh = plsc.VectorSubcoreMesh(core_axis_name="c", subcore_axis_name="s")

    @functools.partial(
        pl.kernel,
        out_type=(
            jax.ShapeDtypeStruct((_NW, h // _NW, w), jnp.float32),
            jax.ShapeDtypeStruct((_NW, n // _NW), jnp.float32),
        ),
        mesh=mesh,
        scratch_types=[
            pltpu.VMEM((n,), jnp.float32),
            pltpu.VMEM((n,), jnp.float32),
            pltpu.VMEM((rows_w, w), jnp.float32),
            pltpu.VMEM((pts_w,), jnp.float32),
        ],
    )
    def sc_k(py_hbm, px_hbm, img_out, pt_out, py_v, px_v, rows_v, ptd_v):
        cid = lax.axis_index("c")
        sid = lax.axis_index("s")
        wid = sid * _NUM_CORES + cid

        pltpu.sync_copy(py_hbm, py_v)
        pltpu.sync_copy(px_hbm, px_v)

        p0 = wid * pts_w
        pyv = py_v[pl.ds(p0, _LANES)]
        pxv = px_v[pl.ds(p0, _LANES)]

        def lattice_d2(v, hi):
            idx = (v + 0.5).astype(jnp.int32)
            idx = jnp.minimum(jnp.maximum(idx, 0), hi)
            d = v - idx.astype(jnp.float32)
            return d * d

        ptd_v[...] = lattice_d2(pyv, h - 1) + lattice_d2(pxv, w - 1)
        pltpu.sync_copy(ptd_v, pt_out.at[wid])

        row0 = wid * rows_w
        iota = lax.iota(jnp.int32, _LANES).astype(jnp.float32)
        yfs = [(row0 + r).astype(jnp.float32) for r in range(rows_w)]
        chunks = n // _LANES

        tile_w = 4
        g0 = 0
        while g0 < groups:
            gw = min(tile_w, groups - g0)
            xv_t = [iota + float((g0 + g) * _LANES) for g in range(gw)]
            init = tuple(jnp.full((_LANES,), 1e30, jnp.float32)
                         for _ in range(rows_w * gw))

            def body(jc, accs, xv_t=xv_t, gw=gw):
                j0 = jc * _LANES
                pyv = py_v[pl.ds(j0, _LANES)]
                pxv = px_v[pl.ds(j0, _LANES)]
                out = list(accs)
                for k in range(_LANES):
                    pyj = pyv[k]
                    pxj = pxv[k]
                    dx2 = []
                    for g in range(gw):
                        dx = xv_t[g] - pxj
                        dx2.append(dx * dx)
                    for r in range(rows_w):
                        dy = yfs[r] - pyj
                        dy2 = dy * dy
                        for g in range(gw):
                            i = r * gw + g
                            out[i] = jnp.minimum(out[i], dx2[g] + dy2)
                return tuple(out)

            accs = lax.fori_loop(0, chunks, body, init)
            for r in range(rows_w):
                for g in range(gw):
                    rows_v[r, pl.ds((g0 + g) * _LANES, _LANES)] = accs[r * gw + g]
            g0 += gw

        pltpu.sync_copy(rows_v, img_out.at[wid])

    return sc_k


def _tc_finish(img_ref, pt_ref, out_ref):
    total = (jnp.sum(jnp.sqrt(img_ref[...]))
             + jnp.sum(jnp.sqrt(pt_ref[...])))
    out_ref[...] = jnp.broadcast_to(total, (1, 1))


def kernel(img_render_points, img_ref):
    pts = img_render_points.reshape(-1, 2).astype(jnp.float32)
    n = pts.shape[0]
    h, w = img_ref.shape
    py = pts[:, 0]
    px = pts[:, 1]

    img_d2, pt_d2 = _sc_min_d2(h, w, n)(py, px)

    loss = pl.pallas_call(
        _tc_finish,
        out_shape=jax.ShapeDtypeStruct((1, 1), jnp.float32),
    )(img_d2, pt_d2.reshape(n // 128, 128))
    return loss[0, 0]


# speedup vs baseline: 2.1459x; 1.2018x over previous
import functools

import jax
import jax.numpy as jnp
from jax import lax
from jax.experimental import pallas as pl
from jax.experimental.pallas import tpu as pltpu
from jax.experimental.pallas import tpu_sc as plsc

_NUM_CORES = 2
_NUM_SUBCORES = 16
_LANES = 16
_NW = _NUM_CORES * _NUM_SUBCORES


def _sqrt_vec(x):
    i = plsc.bitcast(x, jnp.int32)
    i = jnp.full((_LANES,), 0x5F3759DF, jnp.int32) - (i >> 1)
    r = plsc.bitcast(i, jnp.float32)
    xh = x * 0.5
    r = r * (1.5 - xh * r * r)
    r = r * (1.5 - xh * r * r)
    return x * r


@functools.lru_cache(maxsize=None)
def _sc_chamfer(h, w, n):
    rows_w = h // _NW
    groups = w // _LANES
    pts_w = n // _NW
    assert rows_w * _NW == h and groups * _LANES == w and pts_w * _NW == n
    assert pts_w == _LANES

    mesh = plsc.VectorSubcoreMesh(core_axis_name="c", subcore_axis_name="s")

    @functools.partial(
        pl.kernel,
        out_type=jax.ShapeDtypeStruct((_NW, _LANES), jnp.float32),
        mesh=mesh,
        compiler_params=pltpu.CompilerParams(needs_layout_passes=False),
        scratch_types=[
            pltpu.VMEM((n,), jnp.float32),
            pltpu.VMEM((n,), jnp.float32),
            pltpu.VMEM((_LANES,), jnp.float32),
        ],
    )
    def sc_k(py_hbm, px_hbm, out, py_v, px_v, part_v):
        cid = lax.axis_index("c")
        sid = lax.axis_index("s")
        wid = sid * _NUM_CORES + cid

        pltpu.sync_copy(py_hbm, py_v)
        pltpu.sync_copy(px_hbm, px_v)

        p0 = wid * pts_w
        pyv = py_v[pl.ds(p0, _LANES)]
        pxv = px_v[pl.ds(p0, _LANES)]

        def lattice_d2(v, hi):
            idx = (v + 0.5).astype(jnp.int32)
            idx = jnp.minimum(jnp.maximum(idx, 0), hi)
            d = v - idx.astype(jnp.float32)
            return d * d

        sum_vec = _sqrt_vec(lattice_d2(pyv, h - 1) + lattice_d2(pxv, w - 1))

        row0 = wid * rows_w
        iota = lax.iota(jnp.int32, _LANES).astype(jnp.float32)
        yfs = [(row0 + r).astype(jnp.float32) for r in range(rows_w)]
        chunks = n // _LANES

        tile_w = 8
        g0 = 0
        while g0 < groups:
            gw = min(tile_w, groups - g0)
            assert gw % 2 == 0
            xv_t = [iota + float((g0 + g) * _LANES) for g in range(gw)]
            big = jnp.full((_LANES,), 1e30, jnp.float32)
            bigb = plsc.pack(big, big, format=plsc.PackFormat.INTERLEAVED)
            init = tuple(bigb for _ in range(rows_w * (gw // 2)))

            def body(jc, accs, xv_t=xv_t, gw=gw):
                j0 = jc * _LANES
                pyv = py_v[pl.ds(j0, _LANES)]
                pxv = px_v[pl.ds(j0, _LANES)]
                out = list(accs)
                for k in range(_LANES):
                    pyj = pyv[k]
                    pxj = pxv[k]
                    packed = []
                    for g in range(gw // 2):
                        dxa = xv_t[2 * g] - pxj
                        dxb = xv_t[2 * g + 1] - pxj
                        packed.append(plsc.pack(dxa * dxa, dxb * dxb,
                                                format=plsc.PackFormat.INTERLEAVED))
                    for r in range(rows_w):
                        dy = yfs[r] - pyj
                        dy2v = jnp.broadcast_to(dy * dy, (_LANES,))
                        dy2b = plsc.pack(dy2v, dy2v,
                                         format=plsc.PackFormat.INTERLEAVED)
                        for g in range(gw // 2):
                            i = r * (gw // 2) + g
                            out[i] = jnp.minimum(out[i], packed[g] + dy2b)
                return tuple(out)

            accs = lax.fori_loop(0, chunks, body, init)
            for a in accs:
                fa, fb = plsc.unpack(a, format=plsc.PackFormat.INTERLEAVED,
                                     preferred_element_type=jnp.float32)
                sum_vec = sum_vec + _sqrt_vec(fa) + _sqrt_vec(fb)
            g0 += gw

        total = jnp.sum(sum_vec)
        part_v[...] = jnp.broadcast_to(total, (_LANES,))
        pltpu.sync_copy(part_v, out.at[wid])

    return sc_k


def kernel(img_render_points, img_ref):
    pts = img_render_points.reshape(-1, 2).astype(jnp.float32)
    n = pts.shape[0]
    h, w = img_ref.shape
    py = pts[:, 0]
    px = pts[:, 1]

    partials = _sc_chamfer(h, w, n)(py, px)
    return jnp.sum(partials[:, 0])
